# Initial kernel scaffold; baseline (speedup 1.0000x reference)
import functools

import jax
import jax.numpy as jnp
from jax import lax
from jax.experimental import pallas as pl
from jax.experimental.pallas import tpu as pltpu
from jax.experimental.pallas import tpu_sc as plsc

D = 128
NSEG = 10000
NSEG_PAD = 10016
NC = 2
NS = 16
NW = NC * NS
SB_ROWS = 256
SLICE = NSEG // NS


def _zero_fill(ref, nrows, ncols):
  z = jnp.zeros((16,), jnp.float32)
  def body(i, c):
    for j in range(ncols // 16):
      ref[i, pl.ds(j * 16, 16)] = z
    return c
  lax.fori_loop(0, nrows, body, 0)


def _scatter_body(data_hbm, ids_hbm, psums_hbm, pcnts_hbm,
                  buf, idb, ones_v, zrow, zcnt, accum, caccum):
  n_superblocks = data_hbm.shape[0] // SB_ROWS
  cid = lax.axis_index("c")
  sid = lax.axis_index("s")
  wid = sid * NC + cid

  _zero_fill(zrow, zrow.shape[0], D)
  _zero_fill(zcnt, zcnt.shape[0], 16)
  one = jnp.ones((16,), jnp.float32)
  def ones_body(i, c):
    ones_v[i, :] = one
    return c
  lax.fori_loop(0, ones_v.shape[0], ones_body, 0)

  zbase = sid * SLICE
  for k in range(SLICE // zrow.shape[0]):
    pltpu.sync_copy(zrow, accum.at[pl.ds(zbase + k * zrow.shape[0],
                                         zrow.shape[0])])
  pltpu.sync_copy(zcnt, caccum.at[pl.ds(zbase, SLICE)])
  plsc.subcore_barrier()

  max_iters = (n_superblocks + NW - 1) // NW
  def sb_body(k, c):
    s = wid + k * NW
    @pl.when(s < n_superblocks)
    def _():
      pltpu.sync_copy(data_hbm.at[pl.ds(s * SB_ROWS, SB_ROWS)], buf)
      pltpu.sync_copy(ids_hbm.at[pl.ds(s * (SB_ROWS // 128), SB_ROWS // 128)],
                      idb)
      for j in range(SB_ROWS // 128):
        pltpu.sync_copy(buf.at[pl.ds(j * 128, 128)],
                        accum.at[idb.at[j]], add=True)
        pltpu.sync_copy(ones_v, caccum.at[idb.at[j]], add=True)
    return c
  lax.fori_loop(0, max_iters, sb_body, 0)
  plsc.subcore_barrier()

  obase = cid * NSEG_PAD + sid * SLICE
  for k in range(SLICE // zrow.shape[0]):
    pltpu.sync_copy(accum.at[pl.ds(sid * SLICE + k * zrow.shape[0],
                                   zrow.shape[0])],
                    psums_hbm.at[pl.ds(obase + k * zrow.shape[0],
                                       zrow.shape[0])])
  pltpu.sync_copy(caccum.at[pl.ds(sid * SLICE, SLICE)],
                  pcnts_hbm.at[pl.ds(obase, SLICE)])


def _merge_body(psums_hbm, pcnts_hbm, out_hbm, pa, pb, ca, cb):
  cid = lax.axis_index("c")
  sid = lax.axis_index("s")
  wid = sid * NC + cid
  rows = NSEG_PAD // NW
  base = wid * rows

  pltpu.sync_copy(psums_hbm.at[pl.ds(base, rows)], pa)
  pltpu.sync_copy(psums_hbm.at[pl.ds(NSEG_PAD + base, rows)], pb)
  pltpu.sync_copy(pcnts_hbm.at[pl.ds(base, rows)], ca)
  pltpu.sync_copy(pcnts_hbm.at[pl.ds(NSEG_PAD + base, rows)], cb)

  one = jnp.ones((16,), jnp.float32)
  def row_body(r, c):
    cnt = ca[r, :] + cb[r, :]
    inv = one / jnp.maximum(cnt, one)
    for j in range(D // 16):
      sl = pl.ds(j * 16, 16)
      pa[r, sl] = (pa[r, sl] + pb[r, sl]) * inv
    return c
  lax.fori_loop(0, rows, row_body, 0)

  pltpu.sync_copy(pa, out_hbm.at[pl.ds(base, rows)])


def kernel(data, segment_ids):
  n = data.shape[0]
  ids2d = segment_ids.astype(jnp.int32).reshape(n // 128, 128)

  mesh = plsc.VectorSubcoreMesh(core_axis_name="c", subcore_axis_name="s",
                                num_cores=NC, num_subcores=NS)

  scatter = pl.kernel(
      _scatter_body,
      out_type=(
          jax.ShapeDtypeStruct((NC * NSEG_PAD, D), jnp.float32),
          jax.ShapeDtypeStruct((NC * NSEG_PAD, 16), jnp.float32),
      ),
      mesh=mesh,
      scratch_types=[
          pltpu.VMEM((SB_ROWS, D), jnp.float32),
          pltpu.VMEM((SB_ROWS // 128, 128), jnp.int32),
          pltpu.VMEM((128, 16), jnp.float32),
          pltpu.VMEM((125, D), jnp.float32),
          pltpu.VMEM((SLICE, 16), jnp.float32),
          pltpu.VMEM_SHARED((NSEG, D), jnp.float32),
          pltpu.VMEM_SHARED((NSEG, 16), jnp.float32),
      ],
  )
  psums, pcnts = scatter(data, ids2d)

  merge = pl.kernel(
      _merge_body,
      out_type=jax.ShapeDtypeStruct((NSEG_PAD, D), jnp.float32),
      mesh=mesh,
      scratch_types=[
          pltpu.VMEM((NSEG_PAD // NW, D), jnp.float32),
          pltpu.VMEM((NSEG_PAD // NW, D), jnp.float32),
          pltpu.VMEM((NSEG_PAD // NW, 16), jnp.float32),
          pltpu.VMEM((NSEG_PAD // NW, 16), jnp.float32),
      ],
  )
  out = merge(psums, pcnts)
  return out[:NSEG]


# speedup vs baseline: 5.0434x; 5.0434x over previous
import jax
import jax.numpy as jnp
from jax import lax
from jax.experimental import pallas as pl
from jax.experimental.pallas import tpu as pltpu
from jax.experimental.pallas import tpu_sc as plsc

D = 128
DH = D // 2
NSEG = 10000
NSEG_PAD = 10240
NC = 2
NS = 16
NW = NC * NS
SB_ROWS = 256
SLICE = NSEG_PAD // NS
CHUNK = 128


def _zero_fill(ref, nrows, ncols):
  z = jnp.zeros((16,), jnp.float32)
  def body(i, c):
    for j in range(ncols // 16):
      ref[i, pl.ds(j * 16, 16)] = z
    return c
  lax.fori_loop(0, nrows, body, 0)


def _scatter_body(data_hbm, ids_hbm, psums_hbm, pcnts_hbm,
                  buf, idb, ones_v, zrow, zcnt, accum, caccum):
  n_superblocks = data_hbm.shape[0] // SB_ROWS
  cid = lax.axis_index("c")
  sid = lax.axis_index("s")

  _zero_fill(zrow, zrow.shape[0], DH)
  _zero_fill(zcnt, zcnt.shape[0], 16)
  one = jnp.ones((16,), jnp.float32)
  def ones_body(i, c):
    ones_v[i, :] = one
    return c
  lax.fori_loop(0, ones_v.shape[0], ones_body, 0)

  zbase = sid * SLICE
  for k in range(SLICE // CHUNK):
    pltpu.sync_copy(zrow, accum.at[pl.ds(zbase + k * CHUNK, CHUNK)])
  pltpu.sync_copy(zcnt, caccum.at[pl.ds(zbase, SLICE)])
  plsc.subcore_barrier()

  max_iters = (n_superblocks + NS - 1) // NS
  def sb_body(k, c):
    s = sid + k * NS
    @pl.when(s < n_superblocks)
    def _():
      pltpu.sync_copy(
          data_hbm.at[pl.ds(s * SB_ROWS, SB_ROWS), pl.ds(cid * DH, DH)], buf)
      pltpu.sync_copy(ids_hbm.at[pl.ds(s * (SB_ROWS // CHUNK),
                                       SB_ROWS // CHUNK)], idb)
      for j in range(SB_ROWS // CHUNK):
        pltpu.sync_copy(buf.at[pl.ds(j * CHUNK, CHUNK)],
                        accum.at[idb.at[j, 0]], add=True)
      @pl.when(s % 2 == cid)
      def _():
        for j in range(SB_ROWS // CHUNK):
          pltpu.sync_copy(ones_v, caccum.at[idb.at[j, 0]], add=True)
    return c
  lax.fori_loop(0, max_iters, sb_body, 0)
  plsc.subcore_barrier()

  obase = cid * NSEG_PAD + sid * SLICE
  for k in range(SLICE // CHUNK):
    pltpu.sync_copy(accum.at[pl.ds(sid * SLICE + k * CHUNK, CHUNK)],
                    psums_hbm.at[pl.ds(obase + k * CHUNK, CHUNK)])
  pltpu.sync_copy(caccum.at[pl.ds(sid * SLICE, SLICE)],
                  pcnts_hbm.at[pl.ds(obase, SLICE)])


def _merge_body(psums_hbm, pcnts_hbm, out_hbm, pa, pb, ca, cb):
  cid = lax.axis_index("c")
  sid = lax.axis_index("s")
  wid = sid * NC + cid
  rows = NSEG_PAD // NW
  base = wid * rows

  pltpu.sync_copy(psums_hbm.at[pl.ds(base, rows)], pa)
  pltpu.sync_copy(psums_hbm.at[pl.ds(NSEG_PAD + base, rows)], pb)
  pltpu.sync_copy(pcnts_hbm.at[pl.ds(base, rows)], ca)
  pltpu.sync_copy(pcnts_hbm.at[pl.ds(NSEG_PAD + base, rows)], cb)

  one = jnp.ones((16,), jnp.float32)
  def row_body(r, c):
    cnt = ca[r, :] + cb[r, :]
    inv = one / jnp.maximum(cnt, one)
    for j in range(DH // 16):
      sl = pl.ds(j * 16, 16)
      pa[r, sl] = pa[r, sl] * inv
      pb[r, sl] = pb[r, sl] * inv
    return c
  lax.fori_loop(0, rows, row_body, 0)

  pltpu.sync_copy(pa, out_hbm.at[pl.ds(base, rows), pl.ds(0, DH)])
  pltpu.sync_copy(pb, out_hbm.at[pl.ds(base, rows), pl.ds(DH, DH)])


def kernel(data, segment_ids):
  n = data.shape[0]
  ids3d = segment_ids.astype(jnp.int32).reshape(n // 128, 1, 128)

  mesh = plsc.VectorSubcoreMesh(core_axis_name="c", subcore_axis_name="s",
                                num_cores=NC, num_subcores=NS)
  params = pltpu.CompilerParams(use_tc_tiling_on_sc=False)

  scatter = pl.kernel(
      _scatter_body,
      compiler_params=params,
      out_type=(
          jax.ShapeDtypeStruct((NC * NSEG_PAD, DH), jnp.float32),
          jax.ShapeDtypeStruct((NC * NSEG_PAD, 16), jnp.float32),
      ),
      mesh=mesh,
      scratch_types=[
          pltpu.VMEM((SB_ROWS, DH), jnp.float32),
          pltpu.VMEM((SB_ROWS // CHUNK, 1, 128), jnp.int32),
          pltpu.VMEM((CHUNK, 16), jnp.float32),
          pltpu.VMEM((CHUNK, DH), jnp.float32),
          pltpu.VMEM((SLICE, 16), jnp.float32),
          pltpu.VMEM_SHARED((NSEG_PAD, DH), jnp.float32),
          pltpu.VMEM_SHARED((NSEG_PAD, 16), jnp.float32),
      ],
  )
  psums, pcnts = scatter(data, ids3d)

  merge = pl.kernel(
      _merge_body,
      compiler_params=params,
      out_type=jax.ShapeDtypeStruct((NSEG_PAD, D), jnp.float32),
      mesh=mesh,
      scratch_types=[
          pltpu.VMEM((NSEG_PAD // NW, DH), jnp.float32),
          pltpu.VMEM((NSEG_PAD // NW, DH), jnp.float32),
          pltpu.VMEM((NSEG_PAD // NW, 16), jnp.float32),
          pltpu.VMEM((NSEG_PAD // NW, 16), jnp.float32),
      ],
  )
  out = merge(psums, pcnts)
  return out[:NSEG]


# speedup vs baseline: 7.9727x; 1.5808x over previous
import jax
import jax.numpy as jnp
from jax import lax
from jax.experimental import pallas as pl
from jax.experimental.pallas import tpu as pltpu
from jax.experimental.pallas import tpu_sc as plsc

D = 128
DH = D // 2
NSEG = 10000
NSEG_PAD = 10240
NC = 2
NS = 16
NW = NC * NS
SB_ROWS = 512
NSB = 625
SBT = (NSB + NS - 1) // NS
SLICE = NSEG_PAD // NS
CHUNK = 128
NCH = SB_ROWS // CHUNK


def _zero_fill(ref, nrows, ncols):
  z = jnp.zeros((16,), jnp.float32)
  def body(i, c):
    for j in range(ncols // 16):
      ref[i, pl.ds(j * 16, 16)] = z
    return c
  lax.fori_loop(0, nrows, body, 0)


def _scatter_body(data_hbm, ids_hbm, psums_hbm, pcnts_hbm,
                  bufa, bufb, idba, idbb, ones_v, zcnt, accum, caccum,
                  fsema, fsemb, ssem):
  cid = lax.axis_index("c")
  sid = lax.axis_index("s")

  s0 = (NSB * sid) // NS
  s_end = (NSB * (sid + 1)) // NS

  _zero_fill(bufa, SB_ROWS, DH)
  _zero_fill(zcnt, zcnt.shape[0], 16)
  one = jnp.ones((16,), jnp.float32)
  def ones_body(i, c):
    ones_v[i, :] = one
    return c
  lax.fori_loop(0, ones_v.shape[0], ones_body, 0)

  zbase = sid * SLICE
  pltpu.sync_copy(bufa, accum.at[pl.ds(zbase, SB_ROWS)])
  pltpu.sync_copy(bufa.at[pl.ds(0, SLICE - SB_ROWS)],
                  accum.at[pl.ds(zbase + SB_ROWS, SLICE - SB_ROWS)])
  pltpu.sync_copy(zcnt, caccum.at[pl.ds(zbase, SLICE)])
  plsc.subcore_barrier()

  col = cid * DH

  def fill(s, buf, idb, sem):
    pltpu.async_copy(
        data_hbm.at[pl.ds(s * SB_ROWS, SB_ROWS), pl.ds(col, DH)], buf, sem)
    pltpu.async_copy(ids_hbm.at[pl.ds(s * NCH, NCH)], idb, sem)

  def fill_wait(s, buf, idb, sem):
    pltpu.make_async_copy(
        data_hbm.at[pl.ds(s * SB_ROWS, SB_ROWS), pl.ds(col, DH)],
        buf, sem).wait()
    pltpu.make_async_copy(ids_hbm.at[pl.ds(s * NCH, NCH)], idb, sem).wait()

  def scatter(s, buf, idb):
    descs = []
    for j in range(NCH):
      descs.append(pltpu.async_copy(
          buf.at[pl.ds(j * CHUNK, CHUNK)], accum.at[idb.at[j, 0]],
          ssem, add=True))
    @pl.when(s % 2 == cid)
    def _():
      cdescs = []
      for j in range(NCH):
        cdescs.append(pltpu.async_copy(
            ones_v, caccum.at[idb.at[j, 0]], ssem, add=True))
      for d in cdescs:
        d.wait()
    for d in descs:
      d.wait()

  @pl.when(s0 < s_end)
  def _():
    fill(s0, bufa, idba, fsema)

  def pipe_body(k, c):
    sa = s0 + 2 * k
    sb = sa + 1
    @pl.when(sb < s_end)
    def _():
      fill(sb, bufb, idbb, fsemb)
    @pl.when(sa < s_end)
    def _():
      fill_wait(sa, bufa, idba, fsema)
      scatter(sa, bufa, idba)
    @pl.when(sa + 2 < s_end)
    def _():
      fill(sa + 2, bufa, idba, fsema)
    @pl.when(sb < s_end)
    def _():
      fill_wait(sb, bufb, idbb, fsemb)
      scatter(sb, bufb, idbb)
    return c
  lax.fori_loop(0, (SBT + 1) // 2, pipe_body, 0)
  plsc.subcore_barrier()

  obase = cid * NSEG_PAD + sid * SLICE
  pltpu.sync_copy(accum.at[pl.ds(sid * SLICE, SLICE)],
                  psums_hbm.at[pl.ds(obase, SLICE)])
  pltpu.sync_copy(caccum.at[pl.ds(sid * SLICE, SLICE)],
                  pcnts_hbm.at[pl.ds(obase, SLICE)])


def _merge_body(psums_hbm, pcnts_hbm, out_hbm, pa, pb, ca, cb):
  cid = lax.axis_index("c")
  sid = lax.axis_index("s")
  wid = sid * NC + cid
  rows = NSEG_PAD // NW
  base = wid * rows

  pltpu.sync_copy(psums_hbm.at[pl.ds(base, rows)], pa)
  pltpu.sync_copy(psums_hbm.at[pl.ds(NSEG_PAD + base, rows)], pb)
  pltpu.sync_copy(pcnts_hbm.at[pl.ds(base, rows)], ca)
  pltpu.sync_copy(pcnts_hbm.at[pl.ds(NSEG_PAD + base, rows)], cb)

  one = jnp.ones((16,), jnp.float32)
  def row_body(r, c):
    cnt = ca[r, :] + cb[r, :]
    inv = one / jnp.maximum(cnt, one)
    for j in range(DH // 16):
      sl = pl.ds(j * 16, 16)
      pa[r, sl] = pa[r, sl] * inv
      pb[r, sl] = pb[r, sl] * inv
    return c
  lax.fori_loop(0, rows, row_body, 0)

  pltpu.sync_copy(pa, out_hbm.at[pl.ds(base, rows), pl.ds(0, DH)])
  pltpu.sync_copy(pb, out_hbm.at[pl.ds(base, rows), pl.ds(DH, DH)])


def kernel(data, segment_ids):
  n = data.shape[0]
  ids3d = segment_ids.astype(jnp.int32).reshape(n // 128, 1, 128)

  mesh = plsc.VectorSubcoreMesh(core_axis_name="c", subcore_axis_name="s",
                                num_cores=NC, num_subcores=NS)
  params = pltpu.CompilerParams(use_tc_tiling_on_sc=False)

  scatter = pl.kernel(
      _scatter_body,
      compiler_params=params,
      out_type=(
          jax.ShapeDtypeStruct((NC * NSEG_PAD, DH), jnp.float32),
          jax.ShapeDtypeStruct((NC * NSEG_PAD, 16), jnp.float32),
      ),
      mesh=mesh,
      scratch_types=[
          pltpu.VMEM((SB_ROWS, DH), jnp.float32),
          pltpu.VMEM((SB_ROWS, DH), jnp.float32),
          pltpu.VMEM((NCH, 1, 128), jnp.int32),
          pltpu.VMEM((NCH, 1, 128), jnp.int32),
          pltpu.VMEM((CHUNK, 16), jnp.float32),
          pltpu.VMEM((SLICE, 16), jnp.float32),
          pltpu.VMEM_SHARED((NSEG_PAD, DH), jnp.float32),
          pltpu.VMEM_SHARED((NSEG_PAD, 16), jnp.float32),
          pltpu.SemaphoreType.DMA,
          pltpu.SemaphoreType.DMA,
          pltpu.SemaphoreType.DMA,
      ],
  )
  psums, pcnts = scatter(data, ids3d)

  merge = pl.kernel(
      _merge_body,
      compiler_params=params,
      out_type=jax.ShapeDtypeStruct((NSEG_PAD, D), jnp.float32),
      mesh=mesh,
      scratch_types=[
          pltpu.VMEM((NSEG_PAD // NW, DH), jnp.float32),
          pltpu.VMEM((NSEG_PAD // NW, DH), jnp.float32),
          pltpu.VMEM((NSEG_PAD // NW, 16), jnp.float32),
          pltpu.VMEM((NSEG_PAD // NW, 16), jnp.float32),
      ],
  )
  out = merge(psums, pcnts)
  return out[:NSEG]
